# Initial kernel scaffold; baseline (speedup 1.0000x reference)
import jax
import jax.numpy as jnp
from jax.experimental import pallas as pl


def kernel(x, edge_index, params):
    raise NotImplementedError("write your pallas kernel here")


# speedup vs baseline: 30.0340x; 30.0340x over previous
import functools

import jax
import jax.numpy as jnp
from jax import lax
from jax.experimental import pallas as pl
from jax.experimental.pallas import tpu as pltpu
from jax.experimental.pallas import tpu_sc as plsc

N = 10000
E = 320000
HID = 128
HEADS = 8
DH = 16
NC = 2
NS = 16
NW = NC * NS
EW = E // NW
CH = 80
NCHUNK = EW // CH
ZCH = 400
NSTRIPE = N // ZCH

_SC_MESH = dict(core_axis_name="c", subcore_axis_name="s",
                num_cores=NC, num_subcores=NS)


def _mlp_in_body(x_ref, w1_ref, b1_ref, g1_ref, c1_ref, w2_ref, b2_ref,
                 g2_ref, c2_ref, o_ref):
    def layer(h, w_ref, b_ref, g_ref, c_ref):
        h = jnp.dot(h, w_ref[...], preferred_element_type=jnp.float32)
        h = h + b_ref[...]
        mu = jnp.mean(h, axis=0)
        d = h - mu
        var = jnp.mean(d * d, axis=0)
        h = d * lax.rsqrt(var + 1e-5) * g_ref[...] + c_ref[...]
        return jnp.maximum(h, 0.0)

    h = layer(x_ref[...], w1_ref, b1_ref, g1_ref, c1_ref)
    o_ref[...] = layer(h, w2_ref, b2_ref, g2_ref, c2_ref)


def _mlp_in(x, p):
    return pl.pallas_call(
        _mlp_in_body,
        out_shape=jax.ShapeDtypeStruct((N, HID), jnp.float32),
    )(x, p['in1']['W'], p['in1']['b'], p['in1_bn']['w'], p['in1_bn']['b'],
      p['in2']['W'], p['in2']['b'], p['in2_bn']['w'], p['in2_bn']['b'])


def _mm_body(h_ref, wl_ref, wr_ref, xl_ref, xr_ref):
    h = h_ref[...]
    xl_ref[...] = jnp.dot(h, wl_ref[...], preferred_element_type=jnp.float32)
    xr_ref[...] = jnp.dot(h, wr_ref[...], preferred_element_type=jnp.float32)


def _mm(h, wl, wr):
    return pl.pallas_call(
        _mm_body,
        out_shape=(jax.ShapeDtypeStruct((N, HID), jnp.float32),
                   jax.ShapeDtypeStruct((N, HID), jnp.float32)),
    )(h, wl, wr)


_EB = 4000


def _edge_body(xl_ref, xr_ref, s_ref, r_ref, m_ref, msg_ref, ex_ref):
    xl = xl_ref[...]
    v = xl + xr_ref[...]
    z = jnp.maximum(v, 0.2 * v)
    logit = jnp.dot(z, s_ref[...], preferred_element_type=jnp.float32, precision=lax.Precision.HIGHEST)
    ex = jnp.exp(logit) * m_ref[...]
    ex_ref[...] = ex
    msg_ref[...] = jnp.dot(ex, r_ref[...],
                           preferred_element_type=jnp.float32, precision=lax.Precision.HIGHEST) * xl


def _edge_math(gxl, gxr, s16, r16, mask16):
    return pl.pallas_call(
        _edge_body,
        grid=(E // _EB,),
        in_specs=[
            pl.BlockSpec((_EB, HID), lambda i: (i, 0)),
            pl.BlockSpec((_EB, HID), lambda i: (i, 0)),
            pl.BlockSpec((HID, 16), lambda i: (0, 0)),
            pl.BlockSpec((16, HID), lambda i: (0, 0)),
            pl.BlockSpec((1, 16), lambda i: (0, 0)),
        ],
        out_specs=(pl.BlockSpec((_EB, HID), lambda i: (i, 0)),
                   pl.BlockSpec((_EB, 16), lambda i: (i, 0))),
        out_shape=(jax.ShapeDtypeStruct((E, HID), jnp.float32),
                   jax.ShapeDtypeStruct((E, 16), jnp.float32)),
    )(gxl, gxr, s16, r16, mask16)


def _post_bn_body(num_ref, den_ref, bias_ref, r_ref, g_ref, c_ref, o_ref):
    num = num_ref[0] + num_ref[1]
    den = jnp.dot(den_ref[0] + den_ref[1], r_ref[...],
                  preferred_element_type=jnp.float32, precision=lax.Precision.HIGHEST)
    h = num / (den + 1e-16) + bias_ref[...]
    mu = jnp.mean(h, axis=0)
    d = h - mu
    var = jnp.mean(d * d, axis=0)
    o_ref[...] = jnp.maximum(d * lax.rsqrt(var + 1e-5) * g_ref[...]
                             + c_ref[...], 0.0)


def _post_bn(num2, den2, bias, r16, g, c):
    return pl.pallas_call(
        _post_bn_body,
        out_shape=jax.ShapeDtypeStruct((N, HID), jnp.float32),
    )(num2, den2, bias, r16, g, c)


def _out_body(num_ref, den_ref, bias_ref, r_ref,
              w1_ref, b1_ref, g1_ref, c1_ref, w2_ref, b2_ref,
              w3_ref, b3_ref, y_ref):
    num = num_ref[0] + num_ref[1]
    den = jnp.dot(den_ref[0] + den_ref[1], r_ref[...],
                  preferred_element_type=jnp.float32, precision=lax.Precision.HIGHEST)
    h = num / (den + 1e-16) + bias_ref[...]
    h = jnp.dot(h, w1_ref[...], preferred_element_type=jnp.float32) + b1_ref[...]
    mu = jnp.mean(h, axis=0)
    d = h - mu
    var = jnp.mean(d * d, axis=0)
    h = jnp.maximum(d * lax.rsqrt(var + 1e-5) * g1_ref[...] + c1_ref[...], 0.0)
    h = jnp.dot(h, w2_ref[...], preferred_element_type=jnp.float32) + b2_ref[...]
    y = jnp.dot(h, w3_ref[...], preferred_element_type=jnp.float32) + b3_ref[...]
    y_ref[...] = y - jnp.mean(y)


def _out_mlp(num2, den2, bias, r16, p):
    return pl.pallas_call(
        _out_body,
        out_shape=jax.ShapeDtypeStruct((N, 1), jnp.float32),
    )(num2, den2, bias, r16,
      p['out1']['W'], p['out1']['b'], p['out1_bn']['w'], p['out1_bn']['b'],
      p['out2']['W'], p['out2']['b'], p['out_lin']['W'], p['out_lin']['b'])


@functools.cache
def _sc_gather_kernel():
    return pl.kernel(
        _sc_gather_body,
        out_type=(jax.ShapeDtypeStruct((E, HID), jnp.float32),
                  jax.ShapeDtypeStruct((E, HID), jnp.float32)),
        mesh=plsc.VectorSubcoreMesh(**_SC_MESH),
        scratch_types=[
            pltpu.VMEM((CH,), jnp.int32),
            pltpu.VMEM((CH,), jnp.int32),
            pltpu.VMEM((CH, HID), jnp.float32),
            pltpu.VMEM((CH, HID), jnp.float32),
            pltpu.SemaphoreType.DMA,
            pltpu.SemaphoreType.DMA,
        ],
        compiler_params=pltpu.CompilerParams(use_tc_tiling_on_sc=False),
    )


def _sc_gather_body(xl_hbm, xr_hbm, src_hbm, dst_hbm, oxl_hbm, oxr_hbm,
                    si_v, di_v, xlr_v, xrr_v, sem1, sem2):
    wid = lax.axis_index("s") * NC + lax.axis_index("c")
    base0 = wid * EW

    def body(g, carry):
        base = pl.multiple_of(base0 + g * CH, CH)
        pltpu.sync_copy(src_hbm.at[pl.ds(base, CH)], si_v)
        pltpu.sync_copy(dst_hbm.at[pl.ds(base, CH)], di_v)
        c1 = pltpu.async_copy(xl_hbm.at[si_v], xlr_v, sem1)
        c2 = pltpu.async_copy(xr_hbm.at[di_v], xrr_v, sem2)
        c1.wait()
        c2.wait()
        pltpu.sync_copy(xlr_v, oxl_hbm.at[pl.ds(base, CH)])
        pltpu.sync_copy(xrr_v, oxr_hbm.at[pl.ds(base, CH)])
        return carry

    lax.fori_loop(0, NCHUNK, body, 0)


@functools.cache
def _sc_scatter_kernel():
    return pl.kernel(
        _sc_scatter_body,
        out_type=(jax.ShapeDtypeStruct((NC, N, HID), jnp.float32),
                  jax.ShapeDtypeStruct((NC, N, 16), jnp.float32)),
        mesh=plsc.VectorSubcoreMesh(**_SC_MESH),
        scratch_types=[
            pltpu.VMEM_SHARED((N, HID), jnp.float32),
            pltpu.VMEM_SHARED((N, 16), jnp.float32),
            pltpu.VMEM((CH,), jnp.int32),
            pltpu.VMEM((CH, HID), jnp.float32),
            pltpu.VMEM((CH, 16), jnp.float32),
        ],
        compiler_params=pltpu.CompilerParams(use_tc_tiling_on_sc=False),
    )


def _sc_scatter_body(msg_hbm, ex_hbm, dst_hbm, num_hbm, den_hbm,
                     num_sh, den_sh, di_v, msg_v, ex_v):
    cid = lax.axis_index("c")
    sid = lax.axis_index("s")
    wid = sid * NC + cid
    base0 = wid * EW

    zero16 = jnp.zeros((16,), jnp.float32)

    def zrow(r, carry):
        for cc in range(HID // 16):
            msg_v[r, pl.ds(cc * 16, 16)] = zero16
        ex_v[r, pl.ds(0, 16)] = zero16
        return carry

    lax.fori_loop(0, CH, zrow, 0)

    def zstripe(i, carry):
        @pl.when(lax.rem(i, NS) == sid)
        def _():
            b = pl.multiple_of(i * CH, CH)
            pltpu.sync_copy(msg_v, num_sh.at[pl.ds(b, CH)])
            pltpu.sync_copy(ex_v, den_sh.at[pl.ds(b, CH)])
        return carry

    lax.fori_loop(0, N // CH, zstripe, 0)

    plsc.subcore_barrier()

    def body(g, carry):
        base = pl.multiple_of(base0 + g * CH, CH)
        pltpu.sync_copy(dst_hbm.at[pl.ds(base, CH)], di_v)
        pltpu.sync_copy(msg_hbm.at[pl.ds(base, CH)], msg_v)
        pltpu.sync_copy(ex_hbm.at[pl.ds(base, CH)], ex_v)
        pltpu.sync_copy(msg_v, num_sh.at[di_v], add=True)
        pltpu.sync_copy(ex_v, den_sh.at[di_v], add=True)
        return carry

    lax.fori_loop(0, NCHUNK, body, 0)

    plsc.subcore_barrier()

    for i in range(NSTRIPE):
        @pl.when(sid == i % NS)
        def _():
            pltpu.sync_copy(num_sh.at[pl.ds(i * ZCH, ZCH)],
                            num_hbm.at[cid, pl.ds(i * ZCH, ZCH)])
            pltpu.sync_copy(den_sh.at[pl.ds(i * ZCH, ZCH)],
                            den_hbm.at[cid, pl.ds(i * ZCH, ZCH)])


def kernel(x, edge_index, params):
    src = edge_index[0]
    dst = edge_index[1]

    cols = jnp.arange(HID)
    heads = cols // DH
    r16 = jnp.zeros((16, HID), jnp.float32).at[heads, cols].set(1.0)
    mask16 = (jnp.arange(16) < HEADS).astype(jnp.float32).reshape(1, 16)

    h = _mlp_in(x, params)
    for i, p in enumerate(params['convs']):
        xl, xr = _mm(h, p['Wl'], p['Wr'])
        gxl, gxr = _sc_gather_kernel()(xl, xr, src, dst)
        s16 = jnp.zeros((HID, 16), jnp.float32).at[cols, heads].set(
            p['att'].reshape(HID))
        msg, ex = _edge_math(gxl, gxr, s16, r16, mask16)
        num2, den2 = _sc_scatter_kernel()(msg, ex, dst)
        if i < NCONV_LAST:
            h = _post_bn(num2, den2, p['bias'], r16, p['bn_w'], p['bn_b'])
        else:
            y = _out_mlp(num2, den2, p['bias'], r16, params)
    return y


NCONV_LAST = 3


# speedup vs baseline: 57.8645x; 1.9266x over previous
import functools

import jax
import jax.numpy as jnp
from jax import lax
from jax.experimental import pallas as pl
from jax.experimental.pallas import tpu as pltpu
from jax.experimental.pallas import tpu_sc as plsc

N = 10000
E = 320000
HID = 128
HEADS = 8
DH = 16
NC = 2
NS = 16
NW = NC * NS
EW = E // NW
CH = 80
NCHUNK = EW // CH
ZCH = 400
NSTRIPE = N // ZCH

_SC_MESH = dict(core_axis_name="c", subcore_axis_name="s",
                num_cores=NC, num_subcores=NS)


def _mlp_in_body(x_ref, w1_ref, b1_ref, g1_ref, c1_ref, w2_ref, b2_ref,
                 g2_ref, c2_ref, o_ref):
    def layer(h, w_ref, b_ref, g_ref, c_ref):
        h = jnp.dot(h, w_ref[...], preferred_element_type=jnp.float32)
        h = h + b_ref[...]
        mu = jnp.mean(h, axis=0)
        d = h - mu
        var = jnp.mean(d * d, axis=0)
        h = d * lax.rsqrt(var + 1e-5) * g_ref[...] + c_ref[...]
        return jnp.maximum(h, 0.0)

    h = layer(x_ref[...], w1_ref, b1_ref, g1_ref, c1_ref)
    o_ref[...] = layer(h, w2_ref, b2_ref, g2_ref, c2_ref)


def _mlp_in(x, p):
    return pl.pallas_call(
        _mlp_in_body,
        out_shape=jax.ShapeDtypeStruct((N, HID), jnp.float32),
    )(x, p['in1']['W'], p['in1']['b'], p['in1_bn']['w'], p['in1_bn']['b'],
      p['in2']['W'], p['in2']['b'], p['in2_bn']['w'], p['in2_bn']['b'])


def _mm_body(h_ref, wl_ref, wr_ref, xl_ref, xr_ref):
    h = h_ref[...]
    xl_ref[...] = jnp.dot(h, wl_ref[...], preferred_element_type=jnp.float32)
    xr_ref[...] = jnp.dot(h, wr_ref[...], preferred_element_type=jnp.float32)


def _mm(h, wl, wr):
    return pl.pallas_call(
        _mm_body,
        out_shape=(jax.ShapeDtypeStruct((N, HID), jnp.float32),
                   jax.ShapeDtypeStruct((N, HID), jnp.float32)),
    )(h, wl, wr)


_EB = 4000


def _edge_body(xl_ref, xr_ref, s_ref, r_ref, m_ref, msg_ref, ex_ref):
    xl = xl_ref[...]
    v = xl + xr_ref[...]
    z = jnp.maximum(v, 0.2 * v)
    logit = jnp.dot(z, s_ref[...], preferred_element_type=jnp.float32, precision=lax.Precision.HIGHEST)
    ex = jnp.exp(logit) * m_ref[...]
    ex_ref[...] = ex
    msg_ref[...] = jnp.dot(ex, r_ref[...],
                           preferred_element_type=jnp.float32, precision=lax.Precision.HIGHEST) * xl


def _edge_math(gxl, gxr, s16, r16, mask16):
    return pl.pallas_call(
        _edge_body,
        grid=(E // _EB,),
        in_specs=[
            pl.BlockSpec((_EB, HID), lambda i: (i, 0)),
            pl.BlockSpec((_EB, HID), lambda i: (i, 0)),
            pl.BlockSpec((HID, 16), lambda i: (0, 0)),
            pl.BlockSpec((16, HID), lambda i: (0, 0)),
            pl.BlockSpec((1, 16), lambda i: (0, 0)),
        ],
        out_specs=(pl.BlockSpec((_EB, HID), lambda i: (i, 0)),
                   pl.BlockSpec((_EB, 16), lambda i: (i, 0))),
        out_shape=(jax.ShapeDtypeStruct((E, HID), jnp.float32),
                   jax.ShapeDtypeStruct((E, 16), jnp.float32)),
    )(gxl, gxr, s16, r16, mask16)


def _post_bn_body(num_ref, den_ref, bias_ref, r_ref, g_ref, c_ref, o_ref):
    num = num_ref[0] + num_ref[1]
    den = jnp.dot(den_ref[0] + den_ref[1], r_ref[...],
                  preferred_element_type=jnp.float32, precision=lax.Precision.HIGHEST)
    h = num / (den + 1e-16) + bias_ref[...]
    mu = jnp.mean(h, axis=0)
    d = h - mu
    var = jnp.mean(d * d, axis=0)
    o_ref[...] = jnp.maximum(d * lax.rsqrt(var + 1e-5) * g_ref[...]
                             + c_ref[...], 0.0)


def _post_bn(num2, den2, bias, r16, g, c):
    return pl.pallas_call(
        _post_bn_body,
        out_shape=jax.ShapeDtypeStruct((N, HID), jnp.float32),
    )(num2, den2, bias, r16, g, c)


def _out_body(num_ref, den_ref, bias_ref, r_ref,
              w1_ref, b1_ref, g1_ref, c1_ref, w2_ref, b2_ref,
              w3_ref, b3_ref, y_ref):
    num = num_ref[0] + num_ref[1]
    den = jnp.dot(den_ref[0] + den_ref[1], r_ref[...],
                  preferred_element_type=jnp.float32, precision=lax.Precision.HIGHEST)
    h = num / (den + 1e-16) + bias_ref[...]
    h = jnp.dot(h, w1_ref[...], preferred_element_type=jnp.float32) + b1_ref[...]
    mu = jnp.mean(h, axis=0)
    d = h - mu
    var = jnp.mean(d * d, axis=0)
    h = jnp.maximum(d * lax.rsqrt(var + 1e-5) * g1_ref[...] + c1_ref[...], 0.0)
    h = jnp.dot(h, w2_ref[...], preferred_element_type=jnp.float32) + b2_ref[...]
    y = jnp.dot(h, w3_ref[...], preferred_element_type=jnp.float32) + b3_ref[...]
    y_ref[...] = y - jnp.mean(y)


def _out_mlp(num2, den2, bias, r16, p):
    return pl.pallas_call(
        _out_body,
        out_shape=jax.ShapeDtypeStruct((N, 1), jnp.float32),
    )(num2, den2, bias, r16,
      p['out1']['W'], p['out1']['b'], p['out1_bn']['w'], p['out1_bn']['b'],
      p['out2']['W'], p['out2']['b'], p['out_lin']['W'], p['out_lin']['b'])


@functools.cache
def _sc_gather_kernel():
    return pl.kernel(
        _sc_gather_body,
        out_type=(jax.ShapeDtypeStruct((E, HID), jnp.float32),
                  jax.ShapeDtypeStruct((E, HID), jnp.float32)),
        mesh=plsc.VectorSubcoreMesh(**_SC_MESH),
        scratch_types=[
            pltpu.VMEM((CH,), jnp.int32),
            pltpu.VMEM((CH,), jnp.int32),
            pltpu.VMEM((CH, HID), jnp.float32),
            pltpu.VMEM((CH, HID), jnp.float32),
            pltpu.SemaphoreType.DMA,
            pltpu.SemaphoreType.DMA,
        ],
        compiler_params=pltpu.CompilerParams(use_tc_tiling_on_sc=False),
    )


def _sc_gather_body(xl_hbm, xr_hbm, src_hbm, dst_hbm, oxl_hbm, oxr_hbm,
                    si_v, di_v, xlr_v, xrr_v, sem1, sem2):
    wid = lax.axis_index("s") * NC + lax.axis_index("c")
    base0 = wid * EW

    def body(g, carry):
        base = pl.multiple_of(base0 + g * CH, CH)
        pltpu.sync_copy(src_hbm.at[pl.ds(base, CH)], si_v)
        pltpu.sync_copy(dst_hbm.at[pl.ds(base, CH)], di_v)
        c1 = pltpu.async_copy(xl_hbm.at[si_v], xlr_v, sem1)
        c2 = pltpu.async_copy(xr_hbm.at[di_v], xrr_v, sem2)
        c1.wait()
        c2.wait()
        pltpu.sync_copy(xlr_v, oxl_hbm.at[pl.ds(base, CH)])
        pltpu.sync_copy(xrr_v, oxr_hbm.at[pl.ds(base, CH)])
        return carry

    lax.fori_loop(0, NCHUNK, body, 0)


@functools.cache
def _sc_scatter_kernel():
    return pl.kernel(
        _sc_scatter_body,
        out_type=(jax.ShapeDtypeStruct((NC, N, HID), jnp.float32),
                  jax.ShapeDtypeStruct((NC, N, 16), jnp.float32)),
        mesh=plsc.VectorSubcoreMesh(**_SC_MESH),
        scratch_types=[
            pltpu.VMEM_SHARED((N, HID), jnp.float32),
            pltpu.VMEM_SHARED((N, 16), jnp.float32),
            pltpu.VMEM((CH,), jnp.int32),
            pltpu.VMEM((CH, HID), jnp.float32),
            pltpu.VMEM((CH, 16), jnp.float32),
        ],
        compiler_params=pltpu.CompilerParams(use_tc_tiling_on_sc=False),
    )


def _sc_scatter_body(msg_hbm, ex_hbm, dst_hbm, num_hbm, den_hbm,
                     num_sh, den_sh, di_v, msg_v, ex_v):
    cid = lax.axis_index("c")
    sid = lax.axis_index("s")
    wid = sid * NC + cid
    base0 = wid * EW

    zero16 = jnp.zeros((16,), jnp.float32)

    def zrow(r, carry):
        for cc in range(HID // 16):
            msg_v[r, pl.ds(cc * 16, 16)] = zero16
        ex_v[r, pl.ds(0, 16)] = zero16
        return carry

    lax.fori_loop(0, CH, zrow, 0)

    def zstripe(i, carry):
        @pl.when(lax.rem(i, NS) == sid)
        def _():
            b = pl.multiple_of(i * CH, CH)
            pltpu.sync_copy(msg_v, num_sh.at[pl.ds(b, CH)])
            pltpu.sync_copy(ex_v, den_sh.at[pl.ds(b, CH)])
        return carry

    lax.fori_loop(0, N // CH, zstripe, 0)

    plsc.subcore_barrier()

    def body(g, carry):
        base = pl.multiple_of(base0 + g * CH, CH)
        pltpu.sync_copy(dst_hbm.at[pl.ds(base, CH)], di_v)
        pltpu.sync_copy(msg_hbm.at[pl.ds(base, CH)], msg_v)
        pltpu.sync_copy(ex_hbm.at[pl.ds(base, CH)], ex_v)
        pltpu.sync_copy(msg_v, num_sh.at[di_v], add=True)
        pltpu.sync_copy(ex_v, den_sh.at[di_v], add=True)
        return carry

    lax.fori_loop(0, NCHUNK, body, 0)

    plsc.subcore_barrier()

    for i in range(NSTRIPE):
        @pl.when(sid == i % NS)
        def _():
            pltpu.sync_copy(num_sh.at[pl.ds(i * ZCH, ZCH)],
                            num_hbm.at[cid, pl.ds(i * ZCH, ZCH)])
            pltpu.sync_copy(den_sh.at[pl.ds(i * ZCH, ZCH)],
                            den_hbm.at[cid, pl.ds(i * ZCH, ZCH)])


@functools.cache
def _sc_edge_kernel():
    return pl.kernel(
        _sc_edge_body,
        out_type=(jax.ShapeDtypeStruct((NC, N, HID), jnp.float32),
                  jax.ShapeDtypeStruct((NC, N, 16), jnp.float32)),
        mesh=plsc.VectorSubcoreMesh(**_SC_MESH),
        scratch_types=[
            pltpu.VMEM_SHARED((N, HID), jnp.float32),
            pltpu.VMEM_SHARED((N, 16), jnp.float32),
            pltpu.VMEM((CH,), jnp.int32),
            pltpu.VMEM((CH,), jnp.int32),
            pltpu.VMEM((CH, HID), jnp.float32),
            pltpu.VMEM((CH, HID), jnp.float32),
            pltpu.VMEM((CH, HID), jnp.float32),
            pltpu.VMEM((CH, 16), jnp.float32),
            pltpu.VMEM((HEADS, 16), jnp.float32),
            pltpu.SemaphoreType.DMA,
            pltpu.SemaphoreType.DMA,
        ],
        compiler_params=pltpu.CompilerParams(use_tc_tiling_on_sc=False,
                                             needs_layout_passes=False),
    )


def _sc_edge_body(xl_hbm, xr_hbm, src_hbm, dst_hbm, att_hbm, num_hbm, den_hbm,
                  num_sh, den_sh, si_v, di_v, xlr_v, xrr_v, msg_v, exb_v,
                  att_v, sem1, sem2):
    cid = lax.axis_index("c")
    sid = lax.axis_index("s")
    wid = sid * NC + cid
    base0 = wid * EW

    pltpu.sync_copy(att_hbm, att_v)

    zero16 = jnp.zeros((16,), jnp.float32)

    def zrow(r, carry):
        for cc in range(HID // 16):
            msg_v[r, pl.ds(cc * 16, 16)] = zero16
        exb_v[r, pl.ds(0, 16)] = zero16
        return carry

    lax.fori_loop(0, CH, zrow, 0)

    def zstripe(i, carry):
        @pl.when(lax.rem(i, NS) == sid)
        def _():
            b = pl.multiple_of(i * CH, CH)
            pltpu.sync_copy(msg_v, num_sh.at[pl.ds(b, CH)])
            pltpu.sync_copy(exb_v, den_sh.at[pl.ds(b, CH)])
        return carry

    lax.fori_loop(0, N // CH, zstripe, 0)

    atts = [att_v[h, pl.ds(0, 16)] for h in range(HEADS)]
    lanes = lax.iota(jnp.int32, 16)

    plsc.subcore_barrier()

    def chunk(g, carry):
        base = pl.multiple_of(base0 + g * CH, CH)
        pltpu.sync_copy(src_hbm.at[pl.ds(base, CH)], si_v)
        pltpu.sync_copy(dst_hbm.at[pl.ds(base, CH)], di_v)
        c1 = pltpu.async_copy(xl_hbm.at[si_v], xlr_v, sem1)
        c2 = pltpu.async_copy(xr_hbm.at[di_v], xrr_v, sem2)
        c1.wait()
        c2.wait()

        def edge(e, carry2):
            exrow = zero16
            for h in range(HEADS):
                xl = xlr_v[e, pl.ds(h * 16, 16)]
                v = xl + xrr_v[e, pl.ds(h * 16, 16)]
                z = jnp.maximum(v, 0.2 * v)
                logit = jnp.sum(z * atts[h])
                exv = jnp.exp(jnp.broadcast_to(logit, (16,)))
                msg_v[e, pl.ds(h * 16, 16)] = exv * xl
                exrow = jnp.where(lanes == h, exv, exrow)
            exb_v[e, pl.ds(0, 16)] = exrow
            return carry2

        lax.fori_loop(0, CH, edge, 0)
        pltpu.sync_copy(msg_v, num_sh.at[di_v], add=True)
        pltpu.sync_copy(exb_v, den_sh.at[di_v], add=True)
        return carry

    lax.fori_loop(0, NCHUNK, chunk, 0)

    plsc.subcore_barrier()

    for i in range(NSTRIPE):
        @pl.when(sid == i % NS)
        def _():
            pltpu.sync_copy(num_sh.at[pl.ds(i * ZCH, ZCH)],
                            num_hbm.at[cid, pl.ds(i * ZCH, ZCH)])
            pltpu.sync_copy(den_sh.at[pl.ds(i * ZCH, ZCH)],
                            den_hbm.at[cid, pl.ds(i * ZCH, ZCH)])


def kernel(x, edge_index, params):
    src = edge_index[0]
    dst = edge_index[1]

    cols = jnp.arange(HID)
    heads = cols // DH
    r16 = jnp.zeros((16, HID), jnp.float32).at[heads, cols].set(1.0)
    mask16 = (jnp.arange(16) < HEADS).astype(jnp.float32).reshape(1, 16)

    h = _mlp_in(x, params)
    for i, p in enumerate(params['convs']):
        xl, xr = _mm(h, p['Wl'], p['Wr'])
        num2, den2 = _sc_edge_kernel()(xl, xr, src, dst, p['att'])
        if i < NCONV_LAST:
            h = _post_bn(num2, den2, p['bias'], r16, p['bn_w'], p['bn_b'])
        else:
            y = _out_mlp(num2, den2, p['bias'], r16, params)
    return y


NCONV_LAST = 3


# speedup vs baseline: 74.6637x; 1.2903x over previous
import functools

import jax
import jax.numpy as jnp
from jax import lax
from jax.experimental import pallas as pl
from jax.experimental.pallas import tpu as pltpu
from jax.experimental.pallas import tpu_sc as plsc

N = 10000
E = 320000
HID = 128
HEADS = 8
DH = 16
NC = 2
NS = 16
NW = NC * NS
EW = E // NW
CH = 40
NCHUNK = EW // CH
CPB = 50
NBLK = NCHUNK // CPB
NPAIR = CPB // 2
ZST = 400
GBYTES = CH * HID * 4
DBYTES = CH * 16 * 4

_SC_MESH = dict(core_axis_name="c", subcore_axis_name="s",
                num_cores=NC, num_subcores=NS)


def _mlp_in_body(x_ref, w1_ref, b1_ref, g1_ref, c1_ref, w2_ref, b2_ref,
                 g2_ref, c2_ref, o_ref):
    def layer(h, w_ref, b_ref, g_ref, c_ref):
        h = jnp.dot(h, w_ref[...], preferred_element_type=jnp.float32)
        h = h + b_ref[...]
        mu = jnp.mean(h, axis=0)
        d = h - mu
        var = jnp.mean(d * d, axis=0)
        h = d * lax.rsqrt(var + 1e-5) * g_ref[...] + c_ref[...]
        return jnp.maximum(h, 0.0)

    h = layer(x_ref[...], w1_ref, b1_ref, g1_ref, c1_ref)
    o_ref[...] = layer(h, w2_ref, b2_ref, g2_ref, c2_ref)


def _mlp_in(x, p):
    return pl.pallas_call(
        _mlp_in_body,
        out_shape=jax.ShapeDtypeStruct((N, HID), jnp.float32),
    )(x, p['in1']['W'], p['in1']['b'], p['in1_bn']['w'], p['in1_bn']['b'],
      p['in2']['W'], p['in2']['b'], p['in2_bn']['w'], p['in2_bn']['b'])


def _mm_body(h_ref, wl_ref, wr_ref, xl_ref, xr_ref):
    h = h_ref[...]
    xl_ref[...] = jnp.dot(h, wl_ref[...], preferred_element_type=jnp.float32)
    xr_ref[...] = jnp.dot(h, wr_ref[...], preferred_element_type=jnp.float32)


def _mm(h, wl, wr):
    return pl.pallas_call(
        _mm_body,
        out_shape=(jax.ShapeDtypeStruct((N, HID), jnp.float32),
                   jax.ShapeDtypeStruct((N, HID), jnp.float32)),
    )(h, wl, wr)


def _post_bn_body(num_ref, den_ref, bias_ref, r_ref, g_ref, c_ref, o_ref):
    num = num_ref[0] + num_ref[1]
    den = jnp.dot(den_ref[0] + den_ref[1], r_ref[...],
                  preferred_element_type=jnp.float32,
                  precision=lax.Precision.HIGHEST)
    h = num / (den + 1e-16) + bias_ref[...]
    mu = jnp.mean(h, axis=0)
    d = h - mu
    var = jnp.mean(d * d, axis=0)
    o_ref[...] = jnp.maximum(d * lax.rsqrt(var + 1e-5) * g_ref[...]
                             + c_ref[...], 0.0)


def _post_bn(num2, den2, bias, r16, g, c):
    return pl.pallas_call(
        _post_bn_body,
        out_shape=jax.ShapeDtypeStruct((N, HID), jnp.float32),
    )(num2, den2, bias, r16, g, c)


def _out_body(num_ref, den_ref, bias_ref, r_ref,
              w1_ref, b1_ref, g1_ref, c1_ref, w2_ref, b2_ref,
              w3_ref, b3_ref, y_ref):
    num = num_ref[0] + num_ref[1]
    den = jnp.dot(den_ref[0] + den_ref[1], r_ref[...],
                  preferred_element_type=jnp.float32,
                  precision=lax.Precision.HIGHEST)
    h = num / (den + 1e-16) + bias_ref[...]
    h = jnp.dot(h, w1_ref[...], preferred_element_type=jnp.float32) + b1_ref[...]
    mu = jnp.mean(h, axis=0)
    d = h - mu
    var = jnp.mean(d * d, axis=0)
    h = jnp.maximum(d * lax.rsqrt(var + 1e-5) * g1_ref[...] + c1_ref[...], 0.0)
    h = jnp.dot(h, w2_ref[...], preferred_element_type=jnp.float32) + b2_ref[...]
    y = jnp.dot(h, w3_ref[...], preferred_element_type=jnp.float32) + b3_ref[...]
    y_ref[...] = y - jnp.mean(y)


def _out_mlp(num2, den2, bias, r16, p):
    return pl.pallas_call(
        _out_body,
        out_shape=jax.ShapeDtypeStruct((N, 1), jnp.float32),
    )(num2, den2, bias, r16,
      p['out1']['W'], p['out1']['b'], p['out1_bn']['w'], p['out1_bn']['b'],
      p['out2']['W'], p['out2']['b'], p['out_lin']['W'], p['out_lin']['b'])


@functools.cache
def _sc_edge_kernel():
    return pl.kernel(
        _sc_edge_body,
        out_type=(jax.ShapeDtypeStruct((NC, N, HID), jnp.float32),
                  jax.ShapeDtypeStruct((NC, N, 16), jnp.float32)),
        mesh=plsc.VectorSubcoreMesh(**_SC_MESH),
        scratch_types=[
            pltpu.VMEM_SHARED((N, HID), jnp.float32),
            pltpu.VMEM_SHARED((N, 16), jnp.float32),
            pltpu.VMEM((CPB, CH), jnp.int32),
            pltpu.VMEM((CPB, CH), jnp.int32),
            pltpu.VMEM((CH, HID), jnp.float32),
            pltpu.VMEM((CH, HID), jnp.float32),
            pltpu.VMEM((CH, HID), jnp.float32),
            pltpu.VMEM((CH, HID), jnp.float32),
            pltpu.VMEM((CH, HID), jnp.float32),
            pltpu.VMEM((CH, HID), jnp.float32),
            pltpu.VMEM((CH, 16), jnp.float32),
            pltpu.VMEM((CH, 16), jnp.float32),
            pltpu.VMEM((HEADS, 16), jnp.float32),
            pltpu.SemaphoreType.DMA,
            pltpu.SemaphoreType.DMA,
            pltpu.SemaphoreType.DMA,
            pltpu.SemaphoreType.DMA,
            pltpu.SemaphoreType.DMA,
            pltpu.SemaphoreType.DMA,
            pltpu.SemaphoreType.DMA,
            pltpu.SemaphoreType.DMA,
        ],
        compiler_params=pltpu.CompilerParams(use_tc_tiling_on_sc=False,
                                             needs_layout_passes=False),
    )


def _sc_edge_body(xl_hbm, xr_hbm, src_hbm, dst_hbm, att_hbm, num_hbm, den_hbm,
                  num_sh, den_sh, si_v, di_v,
                  xlr0, xlr1, xrr0, xrr1, msg0, msg1, exb0, exb1, att_v,
                  sxl0, sxl1, sxr0, sxr1, snum0, snum1, sden0, sden1):
    cid = lax.axis_index("c")
    sid = lax.axis_index("s")
    wid = sid * NC + cid
    rowbase = wid * NCHUNK

    xlr = (xlr0, xlr1)
    xrr = (xrr0, xrr1)
    msg = (msg0, msg1)
    exb = (exb0, exb1)
    sxl = (sxl0, sxl1)
    sxr = (sxr0, sxr1)
    snum = (snum0, snum1)
    sden = (sden0, sden1)

    pltpu.sync_copy(att_hbm, att_v)

    zero16 = jnp.zeros((16,), jnp.float32)

    def zrow(r, carry):
        for cc in range(HID // 16):
            msg0[r, pl.ds(cc * 16, 16)] = zero16
        exb0[r, pl.ds(0, 16)] = zero16
        return carry

    lax.fori_loop(0, CH, zrow, 0)

    def zstripe(i, carry):
        @pl.when(lax.rem(i, NS) == sid)
        def _():
            b = pl.multiple_of(i * CH, CH)
            pltpu.sync_copy(msg0, num_sh.at[pl.ds(b, CH)])
            pltpu.sync_copy(exb0, den_sh.at[pl.ds(b, CH)])
        return carry

    lax.fori_loop(0, N // CH, zstripe, 0)

    atts = [att_v[h, pl.ds(0, 16)] for h in range(HEADS)]
    lanes = lax.iota(jnp.int32, 16)

    def start_gather(buf, j):
        pltpu.async_copy(xl_hbm.at[si_v.at[j]], xlr[buf], sxl[buf])
        pltpu.async_copy(xr_hbm.at[di_v.at[j]], xrr[buf], sxr[buf])

    def drain_gather(buf):
        pltpu.make_async_copy(xl_hbm.at[pl.ds(0, CH)], xlr[buf], sxl[buf]).wait()
        pltpu.make_async_copy(xr_hbm.at[pl.ds(0, CH)], xrr[buf], sxr[buf]).wait()

    def start_scatter(buf, j):
        pltpu.async_copy(msg[buf], num_sh.at[di_v.at[j]], snum[buf], add=True)
        pltpu.async_copy(exb[buf], den_sh.at[di_v.at[j]], sden[buf], add=True)

    def drain_scatter(buf):
        pltpu.make_async_copy(msg[buf], num_sh.at[pl.ds(0, CH)],
                              snum[buf]).wait()
        pltpu.make_async_copy(exb[buf], den_sh.at[pl.ds(0, CH)],
                              sden[buf]).wait()

    def compute(buf):
        xlr_v, xrr_v, msg_v, exb_v = xlr[buf], xrr[buf], msg[buf], exb[buf]

        @plsc.parallel_loop(0, CH, unroll=2)
        def _(e):
            exrow = zero16
            for h in range(HEADS):
                xl = xlr_v[e, pl.ds(h * 16, 16)]
                v = xl + xrr_v[e, pl.ds(h * 16, 16)]
                z = jnp.maximum(v, 0.2 * v)
                logit = jnp.sum(z * atts[h])
                exv = jnp.exp(jnp.broadcast_to(logit, (16,)))
                msg_v[e, pl.ds(h * 16, 16)] = exv * xl
                exrow = jnp.where(lanes == h, exv, exrow)
            exb_v[e, pl.ds(0, 16)] = exrow

    plsc.subcore_barrier()

    def block(b, carry):
        rb = pl.multiple_of(rowbase + b * CPB, CPB)
        pltpu.sync_copy(src_hbm.at[pl.ds(rb, CPB)], si_v)
        pltpu.sync_copy(dst_hbm.at[pl.ds(rb, CPB)], di_v)
        start_gather(0, 0)

        def pair(jj, carry2):
            ja = 2 * jj
            start_gather(1, ja + 1)
            drain_gather(0)

            @pl.when(jj > 0)
            def _():
                drain_scatter(0)

            compute(0)
            start_scatter(0, ja)

            @pl.when(jj > 0)
            def _():
                drain_scatter(1)

            @pl.when(jj < NPAIR - 1)
            def _():
                start_gather(0, ja + 2)

            drain_gather(1)
            compute(1)
            start_scatter(1, ja + 1)
            return carry2

        lax.fori_loop(0, NPAIR, pair, 0)
        drain_scatter(0)
        drain_scatter(1)
        return carry

    lax.fori_loop(0, NBLK, block, 0)

    plsc.subcore_barrier()

    for i in range(N // ZST):
        @pl.when(sid == i % NS)
        def _():
            pltpu.sync_copy(num_sh.at[pl.ds(i * ZST, ZST)],
                            num_hbm.at[cid, pl.ds(i * ZST, ZST)])
            pltpu.sync_copy(den_sh.at[pl.ds(i * ZST, ZST)],
                            den_hbm.at[cid, pl.ds(i * ZST, ZST)])


def kernel(x, edge_index, params):
    src2 = edge_index[0].reshape(E // CH, CH)
    dst2 = edge_index[1].reshape(E // CH, CH)

    cols = jnp.arange(HID)
    heads = cols // DH
    r16 = jnp.zeros((16, HID), jnp.float32).at[heads, cols].set(1.0)

    h = _mlp_in(x, params)
    for i, p in enumerate(params['convs']):
        xl, xr = _mm(h, p['Wl'], p['Wr'])
        num2, den2 = _sc_edge_kernel()(xl, xr, src2, dst2, p['att'])
        if i < len(params['convs']) - 1:
            h = _post_bn(num2, den2, p['bias'], r16, p['bn_w'], p['bn_b'])
        else:
            y = _out_mlp(num2, den2, p['bias'], r16, params)
    return y


# speedup vs baseline: 95.1755x; 1.2747x over previous
import functools

import jax
import jax.numpy as jnp
from jax import lax
from jax.experimental import pallas as pl
from jax.experimental.pallas import tpu as pltpu
from jax.experimental.pallas import tpu_sc as plsc

N = 10000
E = 320000
HID = 128
HEADS = 8
DH = 16
NC = 2
NS = 16
NW = NC * NS
EW = E // NW
CH = 40
NCHUNK = EW // CH
CPB = 50
NBLK = NCHUNK // CPB
NPAIR = CPB // 2
ZST = 400
GBYTES = CH * HID * 4
DBYTES = CH * 16 * 4

_SC_MESH = dict(core_axis_name="c", subcore_axis_name="s",
                num_cores=NC, num_subcores=NS)


def _mlp_in_body(x_ref, w1_ref, b1_ref, g1_ref, c1_ref, w2_ref, b2_ref,
                 g2_ref, c2_ref, o_ref):
    def layer(h, w_ref, b_ref, g_ref, c_ref):
        h = jnp.dot(h, w_ref[...], preferred_element_type=jnp.float32)
        h = h + b_ref[...]
        mu = jnp.mean(h, axis=0)
        d = h - mu
        var = jnp.mean(d * d, axis=0)
        h = d * lax.rsqrt(var + 1e-5) * g_ref[...] + c_ref[...]
        return jnp.maximum(h, 0.0)

    h = layer(x_ref[...], w1_ref, b1_ref, g1_ref, c1_ref)
    o_ref[...] = layer(h, w2_ref, b2_ref, g2_ref, c2_ref)


def _mlp_in(x, p):
    return pl.pallas_call(
        _mlp_in_body,
        out_shape=jax.ShapeDtypeStruct((N, HID), jnp.float32),
    )(x, p['in1']['W'], p['in1']['b'], p['in1_bn']['w'], p['in1_bn']['b'],
      p['in2']['W'], p['in2']['b'], p['in2_bn']['w'], p['in2_bn']['b'])


def _mm_body(h_ref, wl_ref, wr_ref, xl_ref, xr_ref):
    h = h_ref[...]
    xl_ref[...] = jnp.dot(h, wl_ref[...], preferred_element_type=jnp.float32)
    xr_ref[...] = jnp.dot(h, wr_ref[...], preferred_element_type=jnp.float32)


def _mm(h, wl, wr):
    return pl.pallas_call(
        _mm_body,
        out_shape=(jax.ShapeDtypeStruct((N, HID), jnp.float32),
                   jax.ShapeDtypeStruct((N, HID), jnp.float32)),
    )(h, wl, wr)


def _post_bn_body(num_ref, den_ref, bias_ref, r_ref, g_ref, c_ref, o_ref):
    num = num_ref[0] + num_ref[1]
    den = jnp.dot(den_ref[0] + den_ref[1], r_ref[...],
                  preferred_element_type=jnp.float32,
                  precision=lax.Precision.HIGHEST)
    h = num / (den + 1e-16) + bias_ref[...]
    mu = jnp.mean(h, axis=0)
    d = h - mu
    var = jnp.mean(d * d, axis=0)
    o_ref[...] = jnp.maximum(d * lax.rsqrt(var + 1e-5) * g_ref[...]
                             + c_ref[...], 0.0)


def _post_bn(num2, den2, bias, r16, g, c):
    return pl.pallas_call(
        _post_bn_body,
        out_shape=jax.ShapeDtypeStruct((N, HID), jnp.float32),
    )(num2, den2, bias, r16, g, c)


def _out_body(num_ref, den_ref, bias_ref, r_ref,
              w1_ref, b1_ref, g1_ref, c1_ref, w2_ref, b2_ref,
              w3_ref, b3_ref, y_ref):
    num = num_ref[0] + num_ref[1]
    den = jnp.dot(den_ref[0] + den_ref[1], r_ref[...],
                  preferred_element_type=jnp.float32,
                  precision=lax.Precision.HIGHEST)
    h = num / (den + 1e-16) + bias_ref[...]
    h = jnp.dot(h, w1_ref[...], preferred_element_type=jnp.float32) + b1_ref[...]
    mu = jnp.mean(h, axis=0)
    d = h - mu
    var = jnp.mean(d * d, axis=0)
    h = jnp.maximum(d * lax.rsqrt(var + 1e-5) * g1_ref[...] + c1_ref[...], 0.0)
    h = jnp.dot(h, w2_ref[...], preferred_element_type=jnp.float32) + b2_ref[...]
    y = jnp.dot(h, w3_ref[...], preferred_element_type=jnp.float32) + b3_ref[...]
    y_ref[...] = y - jnp.mean(y)


def _out_mlp(num2, den2, bias, r16, p):
    return pl.pallas_call(
        _out_body,
        out_shape=jax.ShapeDtypeStruct((N, 1), jnp.float32),
    )(num2, den2, bias, r16,
      p['out1']['W'], p['out1']['b'], p['out1_bn']['w'], p['out1_bn']['b'],
      p['out2']['W'], p['out2']['b'], p['out_lin']['W'], p['out_lin']['b'])


@functools.cache
def _sc_edge_kernel():
    return pl.kernel(
        _sc_edge_body,
        out_type=(jax.ShapeDtypeStruct((NC, N, HID), jnp.float32),
                  jax.ShapeDtypeStruct((NC, N, 16), jnp.float32)),
        mesh=plsc.VectorSubcoreMesh(**_SC_MESH),
        scratch_types=[
            pltpu.VMEM_SHARED((N, HID), jnp.float32),
            pltpu.VMEM_SHARED((N, 16), jnp.float32),
            pltpu.VMEM((CPB, CH), jnp.int32),
            pltpu.VMEM((CPB, CH), jnp.int32),
            pltpu.VMEM((CH, HID), jnp.float32),
            pltpu.VMEM((CH, HID), jnp.float32),
            pltpu.VMEM((CH, HID), jnp.float32),
            pltpu.VMEM((CH, HID), jnp.float32),
            pltpu.VMEM((CH, HID), jnp.float32),
            pltpu.VMEM((CH, HID), jnp.float32),
            pltpu.VMEM((CH, 16), jnp.float32),
            pltpu.VMEM((CH, 16), jnp.float32),
            pltpu.VMEM((HEADS, 16), jnp.float32),
            pltpu.SemaphoreType.DMA,
            pltpu.SemaphoreType.DMA,
            pltpu.SemaphoreType.DMA,
            pltpu.SemaphoreType.DMA,
            pltpu.SemaphoreType.DMA,
            pltpu.SemaphoreType.DMA,
            pltpu.SemaphoreType.DMA,
            pltpu.SemaphoreType.DMA,
        ],
        compiler_params=pltpu.CompilerParams(use_tc_tiling_on_sc=False,
                                             needs_layout_passes=False),
    )


def _sc_edge_body(xl_hbm, xr_hbm, src_hbm, dst_hbm, att_hbm, num_hbm, den_hbm,
                  num_sh, den_sh, si_v, di_v,
                  xlr0, xlr1, xrr0, xrr1, msg0, msg1, exb0, exb1, att_v,
                  sxl0, sxl1, sxr0, sxr1, snum0, snum1, sden0, sden1):
    cid = lax.axis_index("c")
    sid = lax.axis_index("s")
    wid = sid * NC + cid
    rowbase = wid * NCHUNK

    xlr = (xlr0, xlr1)
    xrr = (xrr0, xrr1)
    msg = (msg0, msg1)
    exb = (exb0, exb1)
    sxl = (sxl0, sxl1)
    sxr = (sxr0, sxr1)
    snum = (snum0, snum1)
    sden = (sden0, sden1)

    pltpu.sync_copy(att_hbm, att_v)

    zero16 = jnp.zeros((16,), jnp.float32)

    def zrow(r, carry):
        for cc in range(HID // 16):
            msg0[r, pl.ds(cc * 16, 16)] = zero16
        exb0[r, pl.ds(0, 16)] = zero16
        return carry

    lax.fori_loop(0, CH, zrow, 0)

    def zstripe(i, carry):
        @pl.when(lax.rem(i, NS) == sid)
        def _():
            b = pl.multiple_of(i * CH, CH)
            pltpu.sync_copy(msg0, num_sh.at[pl.ds(b, CH)])
            pltpu.sync_copy(exb0, den_sh.at[pl.ds(b, CH)])
        return carry

    lax.fori_loop(0, N // CH, zstripe, 0)

    atts = [att_v[h, pl.ds(0, 16)] for h in range(HEADS)]
    lanes = lax.iota(jnp.int32, 16)

    def start_gather(buf, j):
        pltpu.async_copy(xl_hbm.at[si_v.at[j]], xlr[buf], sxl[buf])
        pltpu.async_copy(xr_hbm.at[di_v.at[j]], xrr[buf], sxr[buf])

    def drain_gather(buf):
        pltpu.make_async_copy(xl_hbm.at[pl.ds(0, CH)], xlr[buf], sxl[buf]).wait()
        pltpu.make_async_copy(xr_hbm.at[pl.ds(0, CH)], xrr[buf], sxr[buf]).wait()

    def start_scatter(buf, j):
        pltpu.async_copy(msg[buf], num_sh.at[di_v.at[j]], snum[buf], add=True)
        pltpu.async_copy(exb[buf], den_sh.at[di_v.at[j]], sden[buf], add=True)

    def drain_scatter(buf):
        pltpu.make_async_copy(msg[buf], num_sh.at[pl.ds(0, CH)],
                              snum[buf]).wait()
        pltpu.make_async_copy(exb[buf], den_sh.at[pl.ds(0, CH)],
                              sden[buf]).wait()

    def compute(buf):
        xlr_v, xrr_v, msg_v, exb_v = xlr[buf], xrr[buf], msg[buf], exb[buf]

        @plsc.parallel_loop(0, CH, unroll=4)
        def _(e):
            exrow = zero16
            for h in range(HEADS):
                xl = xlr_v[e, pl.ds(h * 16, 16)]
                v = xl + xrr_v[e, pl.ds(h * 16, 16)]
                z = jnp.maximum(v, 0.2 * v)
                logit = jnp.sum(z * atts[h])
                exv = jnp.exp(jnp.broadcast_to(logit, (16,)))
                msg_v[e, pl.ds(h * 16, 16)] = exv * xl
                exrow = jnp.where(lanes == h, exv, exrow)
            exb_v[e, pl.ds(0, 16)] = exrow

    plsc.subcore_barrier()

    def block(b, carry):
        rb = pl.multiple_of(rowbase + b * CPB, CPB)
        pltpu.sync_copy(src_hbm.at[pl.ds(rb, CPB)], si_v)
        pltpu.sync_copy(dst_hbm.at[pl.ds(rb, CPB)], di_v)
        start_gather(0, 0)

        def pair(jj, carry2):
            ja = 2 * jj
            start_gather(1, ja + 1)
            drain_gather(0)

            @pl.when(jj > 0)
            def _():
                drain_scatter(0)

            compute(0)
            start_scatter(0, ja)

            @pl.when(jj > 0)
            def _():
                drain_scatter(1)

            @pl.when(jj < NPAIR - 1)
            def _():
                start_gather(0, ja + 2)

            drain_gather(1)
            compute(1)
            start_scatter(1, ja + 1)
            return carry2

        lax.fori_loop(0, NPAIR, pair, 0)
        drain_scatter(0)
        drain_scatter(1)
        return carry

    lax.fori_loop(0, NBLK, block, 0)

    plsc.subcore_barrier()

    for i in range(N // ZST):
        @pl.when(sid == i % NS)
        def _():
            pltpu.sync_copy(num_sh.at[pl.ds(i * ZST, ZST)],
                            num_hbm.at[cid, pl.ds(i * ZST, ZST)])
            pltpu.sync_copy(den_sh.at[pl.ds(i * ZST, ZST)],
                            den_hbm.at[cid, pl.ds(i * ZST, ZST)])


def kernel(x, edge_index, params):
    src2 = edge_index[0].reshape(E // CH, CH)
    dst2 = edge_index[1].reshape(E // CH, CH)

    cols = jnp.arange(HID)
    heads = cols // DH
    r16 = jnp.zeros((16, HID), jnp.float32).at[heads, cols].set(1.0)

    h = _mlp_in(x, params)
    for i, p in enumerate(params['convs']):
        xl, xr = _mm(h, p['Wl'], p['Wr'])
        num2, den2 = _sc_edge_kernel()(xl, xr, src2, dst2, p['att'])
        if i < len(params['convs']) - 1:
            h = _post_bn(num2, den2, p['bias'], r16, p['bn_w'], p['bn_b'])
        else:
            y = _out_mlp(num2, den2, p['bias'], r16, params)
    return y


# speedup vs baseline: 141.1509x; 1.4831x over previous
import functools

import jax
import jax.numpy as jnp
from jax import lax
from jax.experimental import pallas as pl
from jax.experimental.pallas import tpu as pltpu
from jax.experimental.pallas import tpu_sc as plsc

N = 10000
E = 320000
HID = 128
HEADS = 8
DH = 16
NC = 2
NS = 16
NW = NC * NS
EW = E // NW
CH = 40
NCHUNK = EW // CH
CPB = 50
NBLK = NCHUNK // CPB
NPAIR = CPB // 2
ZST = 400
GBYTES = CH * HID * 4
DBYTES = CH * 16 * 4

_SC_MESH = dict(core_axis_name="c", subcore_axis_name="s",
                num_cores=NC, num_subcores=NS)


def _mlp_in_body(x_ref, w1_ref, b1_ref, g1_ref, c1_ref, w2_ref, b2_ref,
                 g2_ref, c2_ref, o_ref):
    def layer(h, w_ref, b_ref, g_ref, c_ref):
        h = jnp.dot(h, w_ref[...], preferred_element_type=jnp.float32)
        h = h + b_ref[...]
        mu = jnp.mean(h, axis=0)
        d = h - mu
        var = jnp.mean(d * d, axis=0)
        h = d * lax.rsqrt(var + 1e-5) * g_ref[...] + c_ref[...]
        return jnp.maximum(h, 0.0)

    h = layer(x_ref[...], w1_ref, b1_ref, g1_ref, c1_ref)
    o_ref[...] = layer(h, w2_ref, b2_ref, g2_ref, c2_ref)


def _mlp_in(x, p):
    return pl.pallas_call(
        _mlp_in_body,
        out_shape=jax.ShapeDtypeStruct((N, HID), jnp.float32),
    )(x, p['in1']['W'], p['in1']['b'], p['in1_bn']['w'], p['in1_bn']['b'],
      p['in2']['W'], p['in2']['b'], p['in2_bn']['w'], p['in2_bn']['b'])


def _mm_body(h_ref, wl_ref, wr_ref, xl_ref, xr_ref):
    h = h_ref[...]
    xl_ref[...] = jnp.dot(h, wl_ref[...], preferred_element_type=jnp.float32)
    xr_ref[...] = jnp.dot(h, wr_ref[...], preferred_element_type=jnp.float32)


def _mm(h, wl, wr):
    return pl.pallas_call(
        _mm_body,
        out_shape=(jax.ShapeDtypeStruct((N, HID), jnp.float32),
                   jax.ShapeDtypeStruct((N, HID), jnp.float32)),
    )(h, wl, wr)


def _post_bn_body(num_ref, den_ref, bias_ref, r_ref, g_ref, c_ref, o_ref):
    num = num_ref[0] + num_ref[1]
    den = jnp.dot(den_ref[0] + den_ref[1], r_ref[...],
                  preferred_element_type=jnp.float32,
                  precision=lax.Precision.HIGHEST)
    h = num / (den + 1e-16) + bias_ref[...]
    mu = jnp.mean(h, axis=0)
    d = h - mu
    var = jnp.mean(d * d, axis=0)
    o_ref[...] = jnp.maximum(d * lax.rsqrt(var + 1e-5) * g_ref[...]
                             + c_ref[...], 0.0)


def _post_bn(num2, den2, bias, r16, g, c):
    return pl.pallas_call(
        _post_bn_body,
        out_shape=jax.ShapeDtypeStruct((N, HID), jnp.float32),
    )(num2, den2, bias, r16, g, c)


def _out_body(num_ref, den_ref, bias_ref, r_ref,
              w1_ref, b1_ref, g1_ref, c1_ref, w2_ref, b2_ref,
              w3_ref, b3_ref, y_ref):
    num = num_ref[0] + num_ref[1]
    den = jnp.dot(den_ref[0] + den_ref[1], r_ref[...],
                  preferred_element_type=jnp.float32,
                  precision=lax.Precision.HIGHEST)
    h = num / (den + 1e-16) + bias_ref[...]
    h = jnp.dot(h, w1_ref[...], preferred_element_type=jnp.float32) + b1_ref[...]
    mu = jnp.mean(h, axis=0)
    d = h - mu
    var = jnp.mean(d * d, axis=0)
    h = jnp.maximum(d * lax.rsqrt(var + 1e-5) * g1_ref[...] + c1_ref[...], 0.0)
    h = jnp.dot(h, w2_ref[...], preferred_element_type=jnp.float32) + b2_ref[...]
    y = jnp.dot(h, w3_ref[...], preferred_element_type=jnp.float32) + b3_ref[...]
    y_ref[...] = y - jnp.mean(y)


def _out_mlp(num2, den2, bias, r16, p):
    return pl.pallas_call(
        _out_body,
        out_shape=jax.ShapeDtypeStruct((N, 1), jnp.float32),
    )(num2, den2, bias, r16,
      p['out1']['W'], p['out1']['b'], p['out1_bn']['w'], p['out1_bn']['b'],
      p['out2']['W'], p['out2']['b'], p['out_lin']['W'], p['out_lin']['b'])


@functools.cache
def _sc_edge_kernel():
    return pl.kernel(
        _sc_edge_body,
        out_type=(jax.ShapeDtypeStruct((NC, N, HID), jnp.float32),
                  jax.ShapeDtypeStruct((NC, N, 16), jnp.float32)),
        mesh=plsc.VectorSubcoreMesh(**_SC_MESH),
        scratch_types=[
            pltpu.VMEM_SHARED((N, HID), jnp.float32),
            pltpu.VMEM_SHARED((N, 16), jnp.float32),
            pltpu.VMEM((CPB, CH), jnp.int32),
            pltpu.VMEM((CPB, CH), jnp.int32),
            pltpu.VMEM((CH, HID), jnp.float32),
            pltpu.VMEM((CH, HID), jnp.float32),
            pltpu.VMEM((CH, HID), jnp.float32),
            pltpu.VMEM((CH, HID), jnp.float32),
            pltpu.VMEM((CH, HID), jnp.float32),
            pltpu.VMEM((CH, HID), jnp.float32),
            pltpu.VMEM((CH, 16), jnp.float32),
            pltpu.VMEM((CH, 16), jnp.float32),
            pltpu.VMEM((HEADS, 16), jnp.float32),
            pltpu.SemaphoreType.DMA,
            pltpu.SemaphoreType.DMA,
            pltpu.SemaphoreType.DMA,
            pltpu.SemaphoreType.DMA,
            pltpu.SemaphoreType.DMA,
            pltpu.SemaphoreType.DMA,
            pltpu.SemaphoreType.DMA,
            pltpu.SemaphoreType.DMA,
        ],
        compiler_params=pltpu.CompilerParams(use_tc_tiling_on_sc=False,
                                             needs_layout_passes=False),
    )


def _sc_edge_body(xl_hbm, xr_hbm, src_hbm, dst_hbm, att_hbm, num_hbm, den_hbm,
                  num_sh, den_sh, si_v, di_v,
                  xlr0, xlr1, xrr0, xrr1, msg0, msg1, exb0, exb1, att_v,
                  sxl0, sxl1, sxr0, sxr1, snum0, snum1, sden0, sden1):
    cid = lax.axis_index("c")
    sid = lax.axis_index("s")
    wid = sid * NC + cid
    rowbase = wid * NCHUNK

    xlr = (xlr0, xlr1)
    xrr = (xrr0, xrr1)
    msg = (msg0, msg1)
    exb = (exb0, exb1)
    sxl = (sxl0, sxl1)
    sxr = (sxr0, sxr1)
    snum = (snum0, snum1)
    sden = (sden0, sden1)

    pltpu.sync_copy(att_hbm, att_v)

    zero16 = jnp.zeros((16,), jnp.float32)

    def zrow(r, carry):
        for cc in range(HID // 16):
            msg0[r, pl.ds(cc * 16, 16)] = zero16
        exb0[r, pl.ds(0, 16)] = zero16
        return carry

    lax.fori_loop(0, CH, zrow, 0)

    def zstripe(i, carry):
        @pl.when(lax.rem(i, NS) == sid)
        def _():
            b = pl.multiple_of(i * CH, CH)
            pltpu.sync_copy(msg0, num_sh.at[pl.ds(b, CH)])
            pltpu.sync_copy(exb0, den_sh.at[pl.ds(b, CH)])
        return carry

    lax.fori_loop(0, N // CH, zstripe, 0)

    atts = [att_v[h, pl.ds(0, 16)] for h in range(HEADS)]
    lanes = lax.iota(jnp.int32, 16)

    def start_gather(buf, j):
        pltpu.async_copy(xl_hbm.at[si_v.at[j]], xlr[buf], sxl[buf])
        pltpu.async_copy(xr_hbm.at[di_v.at[j]], xrr[buf], sxr[buf])

    def drain_gather(buf):
        pltpu.make_async_copy(xl_hbm.at[pl.ds(0, CH)], xlr[buf], sxl[buf]).wait()
        pltpu.make_async_copy(xr_hbm.at[pl.ds(0, CH)], xrr[buf], sxr[buf]).wait()

    def start_scatter(buf, j):
        pltpu.async_copy(msg[buf], num_sh.at[di_v.at[j]], snum[buf], add=True)
        pltpu.async_copy(exb[buf], den_sh.at[di_v.at[j]], sden[buf], add=True)

    def drain_scatter(buf):
        pltpu.make_async_copy(msg[buf], num_sh.at[pl.ds(0, CH)],
                              snum[buf]).wait()
        pltpu.make_async_copy(exb[buf], den_sh.at[pl.ds(0, CH)],
                              sden[buf]).wait()

    def compute(buf):
        xlr_v, xrr_v, msg_v, exb_v = xlr[buf], xrr[buf], msg[buf], exb[buf]

        @plsc.parallel_loop(0, CH, unroll=8)
        def _(e):
            exrow = zero16
            for h in range(HEADS):
                xl = xlr_v[e, pl.ds(h * 16, 16)]
                v = xl + xrr_v[e, pl.ds(h * 16, 16)]
                z = jnp.maximum(v, 0.2 * v)
                logit = jnp.sum(z * atts[h])
                exv = jnp.exp(jnp.broadcast_to(logit, (16,)))
                msg_v[e, pl.ds(h * 16, 16)] = exv * xl
                exrow = jnp.where(lanes == h, exv, exrow)
            exb_v[e, pl.ds(0, 16)] = exrow

    plsc.subcore_barrier()

    def block(b, carry):
        rb = pl.multiple_of(rowbase + b * CPB, CPB)
        pltpu.sync_copy(src_hbm.at[pl.ds(rb, CPB)], si_v)
        pltpu.sync_copy(dst_hbm.at[pl.ds(rb, CPB)], di_v)
        start_gather(0, 0)

        def pair(jj, carry2):
            ja = 2 * jj
            start_gather(1, ja + 1)
            drain_gather(0)

            @pl.when(jj > 0)
            def _():
                drain_scatter(0)

            compute(0)
            start_scatter(0, ja)

            @pl.when(jj > 0)
            def _():
                drain_scatter(1)

            @pl.when(jj < NPAIR - 1)
            def _():
                start_gather(0, ja + 2)

            drain_gather(1)
            compute(1)
            start_scatter(1, ja + 1)
            return carry2

        lax.fori_loop(0, NPAIR, pair, 0)
        drain_scatter(0)
        drain_scatter(1)
        return carry

    lax.fori_loop(0, NBLK, block, 0)

    plsc.subcore_barrier()

    for i in range(N // ZST):
        @pl.when(sid == i % NS)
        def _():
            pltpu.sync_copy(num_sh.at[pl.ds(i * ZST, ZST)],
                            num_hbm.at[cid, pl.ds(i * ZST, ZST)])
            pltpu.sync_copy(den_sh.at[pl.ds(i * ZST, ZST)],
                            den_hbm.at[cid, pl.ds(i * ZST, ZST)])


def kernel(x, edge_index, params):
    src2 = edge_index[0].reshape(E // CH, CH)
    dst2 = edge_index[1].reshape(E // CH, CH)

    cols = jnp.arange(HID)
    heads = cols // DH
    r16 = jnp.zeros((16, HID), jnp.float32).at[heads, cols].set(1.0)

    h = _mlp_in(x, params)
    for i, p in enumerate(params['convs']):
        xl, xr = _mm(h, p['Wl'], p['Wr'])
        num2, den2 = _sc_edge_kernel()(xl, xr, src2, dst2, p['att'])
        if i < len(params['convs']) - 1:
            h = _post_bn(num2, den2, p['bias'], r16, p['bn_w'], p['bn_b'])
        else:
            y = _out_mlp(num2, den2, p['bias'], r16, params)
    return y


# speedup vs baseline: 141.3205x; 1.0012x over previous
import functools

import jax
import jax.numpy as jnp
from jax import lax
from jax.experimental import pallas as pl
from jax.experimental.pallas import tpu as pltpu
from jax.experimental.pallas import tpu_sc as plsc

N = 10000
E = 320000
HID = 128
HEADS = 8
DH = 16
NC = 2
NS = 16
NW = NC * NS
EW = E // NW
CH = 40
NCHUNK = EW // CH
CPB = 50
NBLK = NCHUNK // CPB
NPAIR = CPB // 2
ZST = 400
GBYTES = CH * HID * 4
DBYTES = CH * 16 * 4

_SC_MESH = dict(core_axis_name="c", subcore_axis_name="s",
                num_cores=NC, num_subcores=NS)


def _bn_relu(h, g, c):
    mu = jnp.mean(h, axis=0)
    d = h - mu
    var = jnp.mean(d * d, axis=0)
    return jnp.maximum(d * lax.rsqrt(var + 1e-5) * g + c, 0.0)


def _pre_body(x_ref, w1_ref, b1_ref, g1_ref, c1_ref, w2_ref, b2_ref,
              g2_ref, c2_ref, wl_ref, wr_ref, xl_ref, xr_ref):
    h = jnp.dot(x_ref[...], w1_ref[...], preferred_element_type=jnp.float32)
    h = _bn_relu(h + b1_ref[...], g1_ref[...], c1_ref[...])
    h = jnp.dot(h, w2_ref[...], preferred_element_type=jnp.float32)
    h = _bn_relu(h + b2_ref[...], g2_ref[...], c2_ref[...])
    xl_ref[...] = jnp.dot(h, wl_ref[...], preferred_element_type=jnp.float32)
    xr_ref[...] = jnp.dot(h, wr_ref[...], preferred_element_type=jnp.float32)


def _pre(x, p, pc):
    return pl.pallas_call(
        _pre_body,
        out_shape=(jax.ShapeDtypeStruct((N, HID), jnp.float32),
                   jax.ShapeDtypeStruct((N, HID), jnp.float32)),
    )(x, p['in1']['W'], p['in1']['b'], p['in1_bn']['w'], p['in1_bn']['b'],
      p['in2']['W'], p['in2']['b'], p['in2_bn']['w'], p['in2_bn']['b'],
      pc['Wl'], pc['Wr'])


def _mid_body(num_ref, den_ref, bias_ref, r_ref, g_ref, c_ref,
              wl_ref, wr_ref, xl_ref, xr_ref):
    num = num_ref[0] + num_ref[1]
    den = jnp.dot(den_ref[0] + den_ref[1], r_ref[...],
                  preferred_element_type=jnp.float32,
                  precision=lax.Precision.HIGHEST)
    h = num / (den + 1e-16) + bias_ref[...]
    h = _bn_relu(h, g_ref[...], c_ref[...])
    xl_ref[...] = jnp.dot(h, wl_ref[...], preferred_element_type=jnp.float32)
    xr_ref[...] = jnp.dot(h, wr_ref[...], preferred_element_type=jnp.float32)


def _mid(num2, den2, bias, r16, g, c, pc):
    return pl.pallas_call(
        _mid_body,
        out_shape=(jax.ShapeDtypeStruct((N, HID), jnp.float32),
                   jax.ShapeDtypeStruct((N, HID), jnp.float32)),
    )(num2, den2, bias, r16, g, c, pc['Wl'], pc['Wr'])


def _out_body(num_ref, den_ref, bias_ref, r_ref,
              w1_ref, b1_ref, g1_ref, c1_ref, w2_ref, b2_ref,
              w3_ref, b3_ref, y_ref):
    num = num_ref[0] + num_ref[1]
    den = jnp.dot(den_ref[0] + den_ref[1], r_ref[...],
                  preferred_element_type=jnp.float32,
                  precision=lax.Precision.HIGHEST)
    h = num / (den + 1e-16) + bias_ref[...]
    h = jnp.dot(h, w1_ref[...], preferred_element_type=jnp.float32) + b1_ref[...]
    mu = jnp.mean(h, axis=0)
    d = h - mu
    var = jnp.mean(d * d, axis=0)
    h = jnp.maximum(d * lax.rsqrt(var + 1e-5) * g1_ref[...] + c1_ref[...], 0.0)
    h = jnp.dot(h, w2_ref[...], preferred_element_type=jnp.float32) + b2_ref[...]
    y = jnp.dot(h, w3_ref[...], preferred_element_type=jnp.float32) + b3_ref[...]
    y_ref[...] = y - jnp.mean(y)


def _out_mlp(num2, den2, bias, r16, p):
    return pl.pallas_call(
        _out_body,
        out_shape=jax.ShapeDtypeStruct((N, 1), jnp.float32),
    )(num2, den2, bias, r16,
      p['out1']['W'], p['out1']['b'], p['out1_bn']['w'], p['out1_bn']['b'],
      p['out2']['W'], p['out2']['b'], p['out_lin']['W'], p['out_lin']['b'])


@functools.cache
def _sc_edge_kernel():
    return pl.kernel(
        _sc_edge_body,
        out_type=(jax.ShapeDtypeStruct((NC, N, HID), jnp.float32),
                  jax.ShapeDtypeStruct((NC, N, 16), jnp.float32)),
        mesh=plsc.VectorSubcoreMesh(**_SC_MESH),
        scratch_types=[
            pltpu.VMEM_SHARED((N, HID), jnp.float32),
            pltpu.VMEM_SHARED((N, 16), jnp.float32),
            pltpu.VMEM((CPB, CH), jnp.int32),
            pltpu.VMEM((CPB, CH), jnp.int32),
            pltpu.VMEM((CH, HID), jnp.float32),
            pltpu.VMEM((CH, HID), jnp.float32),
            pltpu.VMEM((CH, HID), jnp.float32),
            pltpu.VMEM((CH, HID), jnp.float32),
            pltpu.VMEM((CH, HID), jnp.float32),
            pltpu.VMEM((CH, HID), jnp.float32),
            pltpu.VMEM((CH, 16), jnp.float32),
            pltpu.VMEM((CH, 16), jnp.float32),
            pltpu.VMEM((HEADS, 16), jnp.float32),
            pltpu.SemaphoreType.DMA,
            pltpu.SemaphoreType.DMA,
            pltpu.SemaphoreType.DMA,
            pltpu.SemaphoreType.DMA,
            pltpu.SemaphoreType.DMA,
            pltpu.SemaphoreType.DMA,
            pltpu.SemaphoreType.DMA,
            pltpu.SemaphoreType.DMA,
        ],
        compiler_params=pltpu.CompilerParams(use_tc_tiling_on_sc=False,
                                             needs_layout_passes=False),
    )


def _sc_edge_body(xl_hbm, xr_hbm, src_hbm, dst_hbm, att_hbm, num_hbm, den_hbm,
                  num_sh, den_sh, si_v, di_v,
                  xlr0, xlr1, xrr0, xrr1, msg0, msg1, exb0, exb1, att_v,
                  sxl0, sxl1, sxr0, sxr1, snum0, snum1, sden0, sden1):
    cid = lax.axis_index("c")
    sid = lax.axis_index("s")
    wid = sid * NC + cid
    rowbase = wid * NCHUNK

    xlr = (xlr0, xlr1)
    xrr = (xrr0, xrr1)
    msg = (msg0, msg1)
    exb = (exb0, exb1)
    sxl = (sxl0, sxl1)
    sxr = (sxr0, sxr1)
    snum = (snum0, snum1)
    sden = (sden0, sden1)

    pltpu.sync_copy(att_hbm, att_v)

    zero16 = jnp.zeros((16,), jnp.float32)

    def zrow(r, carry):
        for cc in range(HID // 16):
            msg0[r, pl.ds(cc * 16, 16)] = zero16
        exb0[r, pl.ds(0, 16)] = zero16
        return carry

    lax.fori_loop(0, CH, zrow, 0)

    def zstripe(i, carry):
        @pl.when(lax.rem(i, NS) == sid)
        def _():
            b = pl.multiple_of(i * CH, CH)
            pltpu.sync_copy(msg0, num_sh.at[pl.ds(b, CH)])
            pltpu.sync_copy(exb0, den_sh.at[pl.ds(b, CH)])
        return carry

    lax.fori_loop(0, N // CH, zstripe, 0)

    atts = [att_v[h, pl.ds(0, 16)] for h in range(HEADS)]
    lanes = lax.iota(jnp.int32, 16)

    def start_gather(buf, j):
        pltpu.async_copy(xl_hbm.at[si_v.at[j]], xlr[buf], sxl[buf])
        pltpu.async_copy(xr_hbm.at[di_v.at[j]], xrr[buf], sxr[buf])

    def drain_gather(buf):
        pltpu.make_async_copy(xl_hbm.at[pl.ds(0, CH)], xlr[buf], sxl[buf]).wait()
        pltpu.make_async_copy(xr_hbm.at[pl.ds(0, CH)], xrr[buf], sxr[buf]).wait()

    def start_scatter(buf, j):
        pltpu.async_copy(msg[buf], num_sh.at[di_v.at[j]], snum[buf], add=True)
        pltpu.async_copy(exb[buf], den_sh.at[di_v.at[j]], sden[buf], add=True)

    def drain_scatter(buf):
        pltpu.make_async_copy(msg[buf], num_sh.at[pl.ds(0, CH)],
                              snum[buf]).wait()
        pltpu.make_async_copy(exb[buf], den_sh.at[pl.ds(0, CH)],
                              sden[buf]).wait()

    def compute(buf):
        xlr_v, xrr_v, msg_v, exb_v = xlr[buf], xrr[buf], msg[buf], exb[buf]

        @plsc.parallel_loop(0, CH, unroll=10)
        def _(e):
            exrow = zero16
            for h in range(HEADS):
                xl = xlr_v[e, pl.ds(h * 16, 16)]
                v = xl + xrr_v[e, pl.ds(h * 16, 16)]
                z = jnp.maximum(v, 0.2 * v)
                logit = jnp.sum(z * atts[h])
                exv = jnp.exp(jnp.broadcast_to(logit, (16,)))
                msg_v[e, pl.ds(h * 16, 16)] = exv * xl
                exrow = jnp.where(lanes == h, exv, exrow)
            exb_v[e, pl.ds(0, 16)] = exrow

    plsc.subcore_barrier()

    def block(b, carry):
        rb = pl.multiple_of(rowbase + b * CPB, CPB)
        pltpu.sync_copy(src_hbm.at[pl.ds(rb, CPB)], si_v)
        pltpu.sync_copy(dst_hbm.at[pl.ds(rb, CPB)], di_v)
        start_gather(0, 0)

        def pair(jj, carry2):
            ja = 2 * jj
            start_gather(1, ja + 1)
            drain_gather(0)

            @pl.when(jj > 0)
            def _():
                drain_scatter(0)

            compute(0)
            start_scatter(0, ja)

            @pl.when(jj > 0)
            def _():
                drain_scatter(1)

            @pl.when(jj < NPAIR - 1)
            def _():
                start_gather(0, ja + 2)

            drain_gather(1)
            compute(1)
            start_scatter(1, ja + 1)
            return carry2

        lax.fori_loop(0, NPAIR, pair, 0)
        drain_scatter(0)
        drain_scatter(1)
        return carry

    lax.fori_loop(0, NBLK, block, 0)

    plsc.subcore_barrier()

    for i in range(N // ZST):
        @pl.when(sid == i % NS)
        def _():
            pltpu.sync_copy(num_sh.at[pl.ds(i * ZST, ZST)],
                            num_hbm.at[cid, pl.ds(i * ZST, ZST)])
            pltpu.sync_copy(den_sh.at[pl.ds(i * ZST, ZST)],
                            den_hbm.at[cid, pl.ds(i * ZST, ZST)])


def kernel(x, edge_index, params):
    src2 = edge_index[0].reshape(E // CH, CH)
    dst2 = edge_index[1].reshape(E // CH, CH)

    cols = jnp.arange(HID)
    heads = cols // DH
    r16 = jnp.zeros((16, HID), jnp.float32).at[heads, cols].set(1.0)

    convs = params['convs']
    xl, xr = _pre(x, params, convs[0])
    for i, p in enumerate(convs):
        num2, den2 = _sc_edge_kernel()(xl, xr, src2, dst2, p['att'])
        if i < len(convs) - 1:
            xl, xr = _mid(num2, den2, p['bias'], r16, p['bn_w'], p['bn_b'],
                          convs[i + 1])
        else:
            y = _out_mlp(num2, den2, p['bias'], r16, params)
    return y


# speedup vs baseline: 142.8481x; 1.0108x over previous
import functools

import jax
import jax.numpy as jnp
from jax import lax
from jax.experimental import pallas as pl
from jax.experimental.pallas import tpu as pltpu
from jax.experimental.pallas import tpu_sc as plsc

N = 10000
E = 320000
HID = 128
HEADS = 8
DH = 16
NC = 2
NS = 16
NW = NC * NS
EW = E // NW
CH = 40
NCHUNK = EW // CH
CPB = 50
NBLK = NCHUNK // CPB
NPAIR = CPB // 2
ZST = 400
GBYTES = CH * HID * 4
DBYTES = CH * 16 * 4

_SC_MESH = dict(core_axis_name="c", subcore_axis_name="s",
                num_cores=NC, num_subcores=NS)


def _bn_relu(h, g, c):
    mu = jnp.mean(h, axis=0)
    d = h - mu
    var = jnp.mean(d * d, axis=0)
    return jnp.maximum(d * lax.rsqrt(var + 1e-5) * g + c, 0.0)


def _pre_body(x_ref, w1_ref, b1_ref, g1_ref, c1_ref, w2_ref, b2_ref,
              g2_ref, c2_ref, wl_ref, wr_ref, xl_ref, xr_ref):
    h = jnp.dot(x_ref[...], w1_ref[...], preferred_element_type=jnp.float32)
    h = _bn_relu(h + b1_ref[...], g1_ref[...], c1_ref[...])
    h = jnp.dot(h, w2_ref[...], preferred_element_type=jnp.float32)
    h = _bn_relu(h + b2_ref[...], g2_ref[...], c2_ref[...])
    xl_ref[...] = jnp.dot(h, wl_ref[...], preferred_element_type=jnp.float32)
    xr_ref[...] = jnp.dot(h, wr_ref[...], preferred_element_type=jnp.float32)


def _pre(x, p, pc):
    return pl.pallas_call(
        _pre_body,
        out_shape=(jax.ShapeDtypeStruct((N, HID), jnp.float32),
                   jax.ShapeDtypeStruct((N, HID), jnp.float32)),
    )(x, p['in1']['W'], p['in1']['b'], p['in1_bn']['w'], p['in1_bn']['b'],
      p['in2']['W'], p['in2']['b'], p['in2_bn']['w'], p['in2_bn']['b'],
      pc['Wl'], pc['Wr'])


def _mid_body(num_ref, den_ref, bias_ref, r_ref, g_ref, c_ref,
              wl_ref, wr_ref, xl_ref, xr_ref):
    num = num_ref[0] + num_ref[1]
    den = jnp.dot(den_ref[0] + den_ref[1], r_ref[...],
                  preferred_element_type=jnp.float32,
                  precision=lax.Precision.HIGHEST)
    h = num / (den + 1e-16) + bias_ref[...]
    h = _bn_relu(h, g_ref[...], c_ref[...])
    xl_ref[...] = jnp.dot(h, wl_ref[...], preferred_element_type=jnp.float32)
    xr_ref[...] = jnp.dot(h, wr_ref[...], preferred_element_type=jnp.float32)


def _mid(num2, den2, bias, r16, g, c, pc):
    return pl.pallas_call(
        _mid_body,
        out_shape=(jax.ShapeDtypeStruct((N, HID), jnp.float32),
                   jax.ShapeDtypeStruct((N, HID), jnp.float32)),
    )(num2, den2, bias, r16, g, c, pc['Wl'], pc['Wr'])


def _out_body(num_ref, den_ref, bias_ref, r_ref,
              w1_ref, b1_ref, g1_ref, c1_ref, w2_ref, b2_ref,
              w3_ref, b3_ref, y_ref):
    num = num_ref[0] + num_ref[1]
    den = jnp.dot(den_ref[0] + den_ref[1], r_ref[...],
                  preferred_element_type=jnp.float32,
                  precision=lax.Precision.HIGHEST)
    h = num / (den + 1e-16) + bias_ref[...]
    h = jnp.dot(h, w1_ref[...], preferred_element_type=jnp.float32) + b1_ref[...]
    mu = jnp.mean(h, axis=0)
    d = h - mu
    var = jnp.mean(d * d, axis=0)
    h = jnp.maximum(d * lax.rsqrt(var + 1e-5) * g1_ref[...] + c1_ref[...], 0.0)
    h = jnp.dot(h, w2_ref[...], preferred_element_type=jnp.float32) + b2_ref[...]
    y = jnp.dot(h, w3_ref[...], preferred_element_type=jnp.float32) + b3_ref[...]
    y_ref[...] = y - jnp.mean(y)


def _out_mlp(num2, den2, bias, r16, p):
    return pl.pallas_call(
        _out_body,
        out_shape=jax.ShapeDtypeStruct((N, 1), jnp.float32),
    )(num2, den2, bias, r16,
      p['out1']['W'], p['out1']['b'], p['out1_bn']['w'], p['out1_bn']['b'],
      p['out2']['W'], p['out2']['b'], p['out_lin']['W'], p['out_lin']['b'])


@functools.cache
def _sc_edge_kernel():
    return pl.kernel(
        _sc_edge_body,
        out_type=(jax.ShapeDtypeStruct((NC, N, HID), jnp.float32),
                  jax.ShapeDtypeStruct((NC, N, 16), jnp.float32)),
        mesh=plsc.VectorSubcoreMesh(**_SC_MESH),
        scratch_types=[
            pltpu.VMEM_SHARED((N, HID), jnp.float32),
            pltpu.VMEM_SHARED((N, 16), jnp.float32),
            pltpu.VMEM((CPB, CH), jnp.int32),
            pltpu.VMEM((CPB, CH), jnp.int32),
            pltpu.VMEM((CH, HID), jnp.float32),
            pltpu.VMEM((CH, HID), jnp.float32),
            pltpu.VMEM((CH, HID), jnp.float32),
            pltpu.VMEM((CH, HID), jnp.float32),
            pltpu.VMEM((CH, HID), jnp.float32),
            pltpu.VMEM((CH, HID), jnp.float32),
            pltpu.VMEM((CH, 16), jnp.float32),
            pltpu.VMEM((CH, 16), jnp.float32),
            pltpu.VMEM((HEADS, 16), jnp.float32),
            pltpu.SemaphoreType.DMA,
            pltpu.SemaphoreType.DMA,
            pltpu.SemaphoreType.DMA,
            pltpu.SemaphoreType.DMA,
            pltpu.SemaphoreType.DMA,
            pltpu.SemaphoreType.DMA,
            pltpu.SemaphoreType.DMA,
            pltpu.SemaphoreType.DMA,
        ],
        compiler_params=pltpu.CompilerParams(use_tc_tiling_on_sc=False,
                                             needs_layout_passes=False),
    )


def _sc_edge_body(xl_hbm, xr_hbm, src_hbm, dst_hbm, att_hbm, num_hbm, den_hbm,
                  num_sh, den_sh, si_v, di_v,
                  xlr0, xlr1, xrr0, xrr1, msg0, msg1, exb0, exb1, att_v,
                  sxl0, sxl1, sxr0, sxr1, snum0, snum1, sden0, sden1):
    cid = lax.axis_index("c")
    sid = lax.axis_index("s")
    wid = sid * NC + cid
    rowbase = wid * NCHUNK

    xlr = (xlr0, xlr1)
    xrr = (xrr0, xrr1)
    msg = (msg0, msg1)
    exb = (exb0, exb1)
    sxl = (sxl0, sxl1)
    sxr = (sxr0, sxr1)
    snum = (snum0, snum1)
    sden = (sden0, sden1)

    pltpu.sync_copy(att_hbm, att_v)

    zero16 = jnp.zeros((16,), jnp.float32)

    def zrow(r, carry):
        for cc in range(HID // 16):
            msg0[r, pl.ds(cc * 16, 16)] = zero16
        exb0[r, pl.ds(0, 16)] = zero16
        return carry

    lax.fori_loop(0, CH, zrow, 0)

    def zstripe(i, carry):
        @pl.when(lax.rem(i, NS) == sid)
        def _():
            b = pl.multiple_of(i * CH, CH)
            pltpu.sync_copy(msg0, num_sh.at[pl.ds(b, CH)])
            pltpu.sync_copy(exb0, den_sh.at[pl.ds(b, CH)])
        return carry

    lax.fori_loop(0, N // CH, zstripe, 0)

    atts = [att_v[h, pl.ds(0, 16)] for h in range(HEADS)]
    lanes = lax.iota(jnp.int32, 16)
    fifteen = jnp.full((16,), 15, jnp.int32)

    def start_gather(buf, j):
        pltpu.async_copy(xl_hbm.at[si_v.at[j]], xlr[buf], sxl[buf])
        pltpu.async_copy(xr_hbm.at[di_v.at[j]], xrr[buf], sxr[buf])

    def drain_gather(buf):
        pltpu.make_async_copy(xl_hbm.at[pl.ds(0, CH)], xlr[buf], sxl[buf]).wait()
        pltpu.make_async_copy(xr_hbm.at[pl.ds(0, CH)], xrr[buf], sxr[buf]).wait()

    def start_scatter(buf, j):
        pltpu.async_copy(msg[buf], num_sh.at[di_v.at[j]], snum[buf], add=True)
        pltpu.async_copy(exb[buf], den_sh.at[di_v.at[j]], sden[buf], add=True)

    def drain_scatter(buf):
        pltpu.make_async_copy(msg[buf], num_sh.at[pl.ds(0, CH)],
                              snum[buf]).wait()
        pltpu.make_async_copy(exb[buf], den_sh.at[pl.ds(0, CH)],
                              sden[buf]).wait()

    def compute(buf):
        xlr_v, xrr_v, msg_v, exb_v = xlr[buf], xrr[buf], msg[buf], exb[buf]

        @plsc.parallel_loop(0, CH, unroll=10)
        def _(e):
            exrow = zero16
            for h in range(HEADS):
                xl = xlr_v[e, pl.ds(h * 16, 16)]
                v = xl + xrr_v[e, pl.ds(h * 16, 16)]
                z = jnp.maximum(v, 0.2 * v)
                csum = jnp.cumsum(z * atts[h])
                exv = jnp.exp(csum).at[fifteen].get(
                    mode="promise_in_bounds")
                msg_v[e, pl.ds(h * 16, 16)] = exv * xl
                exrow = jnp.where(lanes == h, exv, exrow)
            exb_v[e, pl.ds(0, 16)] = exrow

    plsc.subcore_barrier()

    def block(b, carry):
        rb = pl.multiple_of(rowbase + b * CPB, CPB)
        pltpu.sync_copy(src_hbm.at[pl.ds(rb, CPB)], si_v)
        pltpu.sync_copy(dst_hbm.at[pl.ds(rb, CPB)], di_v)
        start_gather(0, 0)

        def pair(jj, carry2):
            ja = 2 * jj
            start_gather(1, ja + 1)
            drain_gather(0)

            @pl.when(jj > 0)
            def _():
                drain_scatter(0)

            compute(0)
            start_scatter(0, ja)

            @pl.when(jj > 0)
            def _():
                drain_scatter(1)

            @pl.when(jj < NPAIR - 1)
            def _():
                start_gather(0, ja + 2)

            drain_gather(1)
            compute(1)
            start_scatter(1, ja + 1)
            return carry2

        lax.fori_loop(0, NPAIR, pair, 0)
        drain_scatter(0)
        drain_scatter(1)
        return carry

    lax.fori_loop(0, NBLK, block, 0)

    plsc.subcore_barrier()

    for i in range(N // ZST):
        @pl.when(sid == i % NS)
        def _():
            pltpu.sync_copy(num_sh.at[pl.ds(i * ZST, ZST)],
                            num_hbm.at[cid, pl.ds(i * ZST, ZST)])
            pltpu.sync_copy(den_sh.at[pl.ds(i * ZST, ZST)],
                            den_hbm.at[cid, pl.ds(i * ZST, ZST)])


def kernel(x, edge_index, params):
    src2 = edge_index[0].reshape(E // CH, CH)
    dst2 = edge_index[1].reshape(E // CH, CH)

    cols = jnp.arange(HID)
    heads = cols // DH
    r16 = jnp.zeros((16, HID), jnp.float32).at[heads, cols].set(1.0)

    convs = params['convs']
    xl, xr = _pre(x, params, convs[0])
    for i, p in enumerate(convs):
        num2, den2 = _sc_edge_kernel()(xl, xr, src2, dst2, p['att'])
        if i < len(convs) - 1:
            xl, xr = _mid(num2, den2, p['bias'], r16, p['bn_w'], p['bn_b'],
                          convs[i + 1])
        else:
            y = _out_mlp(num2, den2, p['bias'], r16, params)
    return y
